# Initial kernel scaffold; baseline (speedup 1.0000x reference)
import jax
import jax.numpy as jnp
from jax.experimental import pallas as pl


def kernel(x, edge_index, ts, c1_w0, c1_w1, c1_w2, c1_w3, c1_b, c2_w0, c2_w1, c2_w2, c2_b, te_w, te_b, fc_w, fc_b):
    raise NotImplementedError("write your pallas kernel here")


# speedup vs baseline: 14.8849x; 14.8849x over previous
import functools

import jax
import jax.numpy as jnp
from jax import lax
from jax.experimental import pallas as pl
from jax.experimental.pallas import tpu as pltpu
from jax.experimental.pallas import tpu_sc as plsc

N = 10000
E = 320000
D = 128
NC = 2
NS = 16
NW = NC * NS
EB = 80


def _mesh():
    return plsc.VectorSubcoreMesh(core_axis_name="c", subcore_axis_name="s")


def _fill(ref, rows, width, value):
    vec = jnp.full((16,), value, dtype=jnp.float32)

    def body(i, _):
        for j in range(width // 16):
            ref[i, pl.ds(j * 16, 16)] = vec
        return 0

    lax.fori_loop(0, rows, body, 0)


RPT = 624
RPT_LAST = N - (NS - 1) * RPT
_ZR = 16


def _zero_acc(zbuf, acc, s):
    def mk(base, nrows):
        def body(i, _):
            pltpu.sync_copy(zbuf, acc.at[pl.ds(base + i * _ZR, _ZR)])
            return 0
        lax.fori_loop(0, nrows // _ZR, body, 0)

    @pl.when(s < NS - 1)
    def _():
        mk(s * RPT, RPT)

    @pl.when(s == NS - 1)
    def _():
        mk((NS - 1) * RPT, RPT_LAST)


def _write_out(acc, out_hbm, c, s):
    @pl.when(s < NS - 1)
    def _():
        pltpu.sync_copy(acc.at[pl.ds(s * RPT, RPT)],
                        out_hbm.at[c, pl.ds(s * RPT, RPT)])

    @pl.when(s == NS - 1)
    def _():
        pltpu.sync_copy(acc.at[pl.ds((NS - 1) * RPT, RPT_LAST)],
                        out_hbm.at[c, pl.ds((NS - 1) * RPT, RPT_LAST)])


DW = 16


@functools.cache
def _get_deg_kernel():
    return functools.partial(
        pl.kernel,
        mesh=_mesh(),
        out_type=jax.ShapeDtypeStruct((NC, N, D), jnp.float32),
        scratch_types=[
            pltpu.VMEM((CHUNK, EB), jnp.int32),
            pltpu.VMEM((EB, D), jnp.float32),
            pltpu.VMEM((_ZR, D), jnp.float32),
            pltpu.VMEM_SHARED((N, D), jnp.float32),
        ],
    )(_deg_body)


def _deg_body(col3_hbm, out_hbm, cidx, ones_v, zbuf, acc):
    c = lax.axis_index("c")
    s = lax.axis_index("s")
    wid = c * NS + s
    _fill(ones_v, EB, D, 1.0)
    _fill(zbuf, _ZR, D, 0.0)
    _zero_acc(zbuf, acc, s)
    plsc.subcore_barrier()

    def chunk(ch, _):
        pltpu.sync_copy(col3_hbm.at[wid, ch], cidx)
        for t in range(CHUNK):
            pltpu.sync_copy(ones_v, acc.at[cidx.at[t]], add=True)
        return 0

    lax.fori_loop(0, NCHUNK, chunk, 0)
    plsc.subcore_barrier()
    _write_out(acc, out_hbm, c, s)


@functools.cache
def _get_prop_kernel():
    return functools.partial(
        pl.kernel,
        mesh=_mesh(),
        out_type=jax.ShapeDtypeStruct((NC, N, D), jnp.float32),
        scratch_types=[
            pltpu.VMEM((EB,), jnp.int32),
            pltpu.VMEM((EB,), jnp.int32),
            pltpu.VMEM((EB, D), jnp.float32),
            pltpu.VMEM((_ZR, D), jnp.float32),
            pltpu.VMEM_SHARED((N, D), jnp.float32),
            pltpu.SemaphoreType.DMA,
        ],
    )(_prop_body)


def _prop_body(w_hbm, row_hbm, col_hbm, out_hbm, ridx, cidx, rows_v, zbuf, acc, sem):
    c = lax.axis_index("c")
    s = lax.axis_index("s")
    wid = c * NS + s
    ept = E // NW
    _fill(zbuf, _ZR, D, 0.0)
    _zero_acc(zbuf, acc, s)
    plsc.subcore_barrier()
    ebase = wid * ept

    def step(i, _):
        off = ebase + i * EB
        pltpu.sync_copy(row_hbm.at[pl.ds(off, EB)], ridx)
        pltpu.sync_copy(col_hbm.at[pl.ds(off, EB)], cidx)
        pltpu.async_copy(w_hbm.at[ridx], rows_v, sem).wait()
        pltpu.sync_copy(rows_v, acc.at[cidx], add=True)
        return 0

    lax.fori_loop(0, ept // EB, step, 0)
    plsc.subcore_barrier()
    _write_out(acc, out_hbm, c, s)


NBLK = E // NW // EB
CHUNK = 25
NCHUNK = NBLK // CHUNK


@functools.cache
def _get_prop2_kernel():
    return functools.partial(
        pl.kernel,
        mesh=_mesh(),
        out_type=jax.ShapeDtypeStruct((NC, N, D), jnp.float32),
        scratch_types=[
            pltpu.VMEM((CHUNK, EB), jnp.int32),
            pltpu.VMEM((CHUNK, EB), jnp.int32),
            pltpu.VMEM((EB, D), jnp.float32),
            pltpu.VMEM((EB, D), jnp.float32),
            pltpu.VMEM((_ZR, D), jnp.float32),
            pltpu.VMEM_SHARED((N, D), jnp.float32),
            pltpu.SemaphoreType.DMA,
            pltpu.SemaphoreType.DMA,
        ],
    )(_prop2_body)


def _prop2_body(w_hbm, row3_hbm, col3_hbm, out_hbm,
                ridx, cidx, rows0, rows1, zbuf, acc, gs0, gs1):
    c = lax.axis_index("c")
    s = lax.axis_index("s")
    wid = c * NS + s
    _fill(zbuf, _ZR, D, 0.0)
    _zero_acc(zbuf, acc, s)
    plsc.subcore_barrier()

    bufs = (rows0, rows1)
    sems = (gs0, gs1)

    def chunk(ch, _):
        pltpu.sync_copy(row3_hbm.at[wid, ch], ridx)
        pltpu.sync_copy(col3_hbm.at[wid, ch], cidx)
        pend = pltpu.async_copy(w_hbm.at[ridx.at[0]], bufs[0], sems[0])
        for t in range(CHUNK):
            nxt = None
            if t < CHUNK - 1:
                nxt = pltpu.async_copy(
                    w_hbm.at[ridx.at[t + 1]],
                    bufs[(t + 1) % 2], sems[(t + 1) % 2])
            pend.wait()
            pltpu.sync_copy(bufs[t % 2], acc.at[cidx.at[t]], add=True)
            pend = nxt
        return 0

    lax.fori_loop(0, NCHUNK, chunk, 0)
    plsc.subcore_barrier()
    _write_out(acc, out_hbm, c, s)


R = 2000
G = N // R
_f32 = jnp.float32


def _dot_t(a, w):
    return lax.dot_general(a, w, (((1,), (1,)), ((), ())),
                           preferred_element_type=_f32,
                           precision=lax.Precision.HIGHEST)


def _prep_body(dp_ref, x_ref, w0_ref, dis_ref, d2_ref):
    deg = dp_ref[0, :, 0:1] + dp_ref[1, :, 0:1]
    dis = jnp.where(deg > 0, lax.rsqrt(jnp.maximum(deg, 1e-12)), 0.0)
    dis_ref[...] = dis
    d2_ref[...] = dis * dis
    w0_ref[...] = x_ref[...] * dis


def _prep(dp, x):
    return pl.pallas_call(
        _prep_body,
        grid=(G,),
        in_specs=[
            pl.BlockSpec((2, R, D), lambda i: (0, i, 0)),
            pl.BlockSpec((R, D), lambda i: (i, 0)),
        ],
        out_specs=[
            pl.BlockSpec((R, D), lambda i: (i, 0)),
            pl.BlockSpec((R, 1), lambda i: (i, 0)),
            pl.BlockSpec((R, 1), lambda i: (i, 0)),
        ],
        out_shape=[
            jax.ShapeDtypeStruct((N, D), _f32),
            jax.ShapeDtypeStruct((N, 1), _f32),
            jax.ShapeDtypeStruct((N, 1), _f32),
        ],
    )(dp, x)


def _step_body(p_ref, dis_ref, d2_ref, w_ref, wn_ref, y_ref):
    b = p_ref[0] + p_ref[1]
    y_ref[...] = _dot_t(b * dis_ref[...], w_ref[...])
    wn_ref[...] = b * d2_ref[...]


def _step(p, dis, d2, w):
    return pl.pallas_call(
        _step_body,
        grid=(G,),
        in_specs=[
            pl.BlockSpec((2, R, D), lambda i: (0, i, 0)),
            pl.BlockSpec((R, 1), lambda i: (i, 0)),
            pl.BlockSpec((R, 1), lambda i: (i, 0)),
            pl.BlockSpec((D, D), lambda i: (0, 0)),
        ],
        out_specs=[
            pl.BlockSpec((R, D), lambda i: (i, 0)),
            pl.BlockSpec((R, D), lambda i: (i, 0)),
        ],
        out_shape=[
            jax.ShapeDtypeStruct((N, D), _f32),
            jax.ShapeDtypeStruct((N, D), _f32),
        ],
    )(p, dis, d2, w)


def _fin1_body(x_ref, w0_ref, y1_ref, y2_ref, y3_ref, b_ref, dis_ref,
               h_ref, wh_ref):
    out = (_dot_t(x_ref[...], w0_ref[...]) + y1_ref[...] + y2_ref[...]
           + y3_ref[...] + b_ref[...])
    h = jnp.maximum(out, 0.0)
    h_ref[...] = h
    wh_ref[...] = h * dis_ref[...]


def _fin1(x, w0, y1, y2, y3, b, dis):
    return pl.pallas_call(
        _fin1_body,
        grid=(G,),
        in_specs=[
            pl.BlockSpec((R, D), lambda i: (i, 0)),
            pl.BlockSpec((D, D), lambda i: (0, 0)),
            pl.BlockSpec((R, D), lambda i: (i, 0)),
            pl.BlockSpec((R, D), lambda i: (i, 0)),
            pl.BlockSpec((R, D), lambda i: (i, 0)),
            pl.BlockSpec((1, D), lambda i: (0, 0)),
            pl.BlockSpec((R, 1), lambda i: (i, 0)),
        ],
        out_specs=[
            pl.BlockSpec((R, D), lambda i: (i, 0)),
            pl.BlockSpec((R, D), lambda i: (i, 0)),
        ],
        out_shape=[
            jax.ShapeDtypeStruct((N, D), _f32),
            jax.ShapeDtypeStruct((N, D), _f32),
        ],
    )(x, w0, y1, y2, y3, b, dis)


def _fin2_body(h_ref, w0_ref, z1_ref, z2_ref, b_ref, ts_ref, tew_ref,
               teb_ref, fwh_ref, fwt_ref, fb_ref, out_ref):
    out2 = jnp.maximum(
        _dot_t(h_ref[...], w0_ref[...]) + z1_ref[...] + z2_ref[...]
        + b_ref[...], 0.0)
    ts_all = ts_ref[...]
    tmin = jnp.min(ts_all)
    tmax = jnp.max(ts_all)
    i = pl.program_id(0)
    tsb = ts_ref[pl.ds(i * R, R), :]
    t = (tsb - tmin) / (tmax - tmin + 1e-8)
    te = t * tew_ref[...] + teb_ref[...]
    res = (lax.dot_general(out2, fwh_ref[...], (((1,), (0,)), ((), ())),
                           preferred_element_type=_f32,
                           precision=lax.Precision.HIGHEST)
           + lax.dot_general(te, fwt_ref[...], (((1,), (0,)), ((), ())),
                             preferred_element_type=_f32,
                             precision=lax.Precision.HIGHEST)
           + fb_ref[...])
    out_ref[...] = res


def _fin2(h, w0, z1, z2, b, ts2, tew, teb, fwh, fwt, fb):
    return pl.pallas_call(
        _fin2_body,
        grid=(G,),
        in_specs=[
            pl.BlockSpec((R, D), lambda i: (i, 0)),
            pl.BlockSpec((D, D), lambda i: (0, 0)),
            pl.BlockSpec((R, D), lambda i: (i, 0)),
            pl.BlockSpec((R, D), lambda i: (i, 0)),
            pl.BlockSpec((1, D), lambda i: (0, 0)),
            pl.BlockSpec((N, 1), lambda i: (0, 0)),
            pl.BlockSpec((1, D), lambda i: (0, 0)),
            pl.BlockSpec((1, D), lambda i: (0, 0)),
            pl.BlockSpec((D, 2), lambda i: (0, 0)),
            pl.BlockSpec((D, 2), lambda i: (0, 0)),
            pl.BlockSpec((1, 2), lambda i: (0, 0)),
        ],
        out_specs=pl.BlockSpec((R, 2), lambda i: (i, 0)),
        out_shape=jax.ShapeDtypeStruct((N, 2), _f32),
    )(h, w0, z1, z2, b, ts2, tew, teb, fwh, fwt, fb)


def kernel(x, edge_index, ts, c1_w0, c1_w1, c1_w2, c1_w3, c1_b,
           c2_w0, c2_w1, c2_w2, c2_b, te_w, te_b, fc_w, fc_b):
    row = edge_index[0]
    col = edge_index[1]

    deg_kernel = _get_deg_kernel()
    prop2 = _get_prop2_kernel()
    row3 = row.reshape(NW, NCHUNK, CHUNK, EB)
    col3 = col.reshape(NW, NCHUNK, CHUNK, EB)

    def prop_kernel(w, _r, _c):
        return prop2(w, row3, col3)

    dp = deg_kernel(col3)
    w0, dis, d2 = _prep(dp, x)

    p1 = prop_kernel(w0, row, col)
    w1, y1 = _step(p1, dis, d2, c1_w1)
    p2 = prop_kernel(w1, row, col)
    w2, y2 = _step(p2, dis, d2, c1_w2)
    p3 = prop_kernel(w2, row, col)
    _, y3 = _step(p3, dis, d2, c1_w3)

    h, wh = _fin1(x, c1_w0, y1, y2, y3, c1_b.reshape(1, D), dis)

    q1 = prop_kernel(wh, row, col)
    v1, z1 = _step(q1, dis, d2, c2_w1)
    q2 = prop_kernel(v1, row, col)
    _, z2 = _step(q2, dis, d2, c2_w2)

    return _fin2(h, c2_w0, z1, z2, c2_b.reshape(1, D), ts.reshape(N, 1),
                 te_w.reshape(1, D), te_b.reshape(1, D),
                 fc_w[:, :D].T, fc_w[:, D:].T, fc_b.reshape(1, 2))


# speedup vs baseline: 16.5752x; 1.1136x over previous
import functools

import jax
import jax.numpy as jnp
from jax import lax
from jax.experimental import pallas as pl
from jax.experimental.pallas import tpu as pltpu
from jax.experimental.pallas import tpu_sc as plsc

N = 10000
E = 320000
D = 128
NC = 2
NS = 16
NW = NC * NS
EB = 80


def _mesh():
    return plsc.VectorSubcoreMesh(core_axis_name="c", subcore_axis_name="s")


def _fill(ref, rows, width, value):
    vec = jnp.full((16,), value, dtype=jnp.float32)

    def body(i, _):
        for j in range(width // 16):
            ref[i, pl.ds(j * 16, 16)] = vec
        return 0

    lax.fori_loop(0, rows, body, 0)


RPT = 624
RPT_LAST = N - (NS - 1) * RPT
_ZR = 16


def _zero_acc(zbuf, acc, s):
    def mk(base, nrows):
        def body(i, _):
            pltpu.sync_copy(zbuf, acc.at[pl.ds(base + i * _ZR, _ZR)])
            return 0
        lax.fori_loop(0, nrows // _ZR, body, 0)

    @pl.when(s < NS - 1)
    def _():
        mk(s * RPT, RPT)

    @pl.when(s == NS - 1)
    def _():
        mk((NS - 1) * RPT, RPT_LAST)


def _zero_acc_hbm(zeros_hbm, acc, s):
    @pl.when(s < NS - 1)
    def _():
        pltpu.sync_copy(zeros_hbm.at[pl.ds(s * RPT, RPT)],
                        acc.at[pl.ds(s * RPT, RPT)])

    @pl.when(s == NS - 1)
    def _():
        pltpu.sync_copy(zeros_hbm.at[pl.ds((NS - 1) * RPT, RPT_LAST)],
                        acc.at[pl.ds((NS - 1) * RPT, RPT_LAST)])


def _write_out(acc, out_hbm, c, s):
    @pl.when(s < NS - 1)
    def _():
        pltpu.sync_copy(acc.at[pl.ds(s * RPT, RPT)],
                        out_hbm.at[c, pl.ds(s * RPT, RPT)])

    @pl.when(s == NS - 1)
    def _():
        pltpu.sync_copy(acc.at[pl.ds((NS - 1) * RPT, RPT_LAST)],
                        out_hbm.at[c, pl.ds((NS - 1) * RPT, RPT_LAST)])


DW = 16


@functools.cache
def _get_deg_kernel():
    return functools.partial(
        pl.kernel,
        mesh=_mesh(),
        out_type=jax.ShapeDtypeStruct((NC, N, D), jnp.float32),
        scratch_types=[
            pltpu.VMEM((CHUNK, EB), jnp.int32),
            pltpu.VMEM((EB, D), jnp.float32),
            pltpu.VMEM_SHARED((N, D), jnp.float32),
            pltpu.SemaphoreType.DMA,
        ],
    )(_deg_body)


def _deg_body(col3_hbm, zeros_hbm, out_hbm, cidx, ones_v, acc, ssem):
    c = lax.axis_index("c")
    s = lax.axis_index("s")
    wid = c * NS + s
    _fill(ones_v, EB, D, 1.0)
    _zero_acc_hbm(zeros_hbm, acc, s)
    plsc.subcore_barrier()

    def chunk(ch, _):
        pltpu.sync_copy(col3_hbm.at[wid, ch], cidx)
        descs = [pltpu.async_copy(ones_v, acc.at[cidx.at[t]], ssem, add=True)
                 for t in range(CHUNK)]
        for dsc in descs:
            dsc.wait()
        return 0

    lax.fori_loop(0, NCHUNK, chunk, 0)
    plsc.subcore_barrier()
    _write_out(acc, out_hbm, c, s)


@functools.cache
def _get_prop_kernel():
    return functools.partial(
        pl.kernel,
        mesh=_mesh(),
        out_type=jax.ShapeDtypeStruct((NC, N, D), jnp.float32),
        scratch_types=[
            pltpu.VMEM((EB,), jnp.int32),
            pltpu.VMEM((EB,), jnp.int32),
            pltpu.VMEM((EB, D), jnp.float32),
            pltpu.VMEM((_ZR, D), jnp.float32),
            pltpu.VMEM_SHARED((N, D), jnp.float32),
            pltpu.SemaphoreType.DMA,
        ],
    )(_prop_body)


def _prop_body(w_hbm, row_hbm, col_hbm, out_hbm, ridx, cidx, rows_v, zbuf, acc, sem):
    c = lax.axis_index("c")
    s = lax.axis_index("s")
    wid = c * NS + s
    ept = E // NW
    _fill(zbuf, _ZR, D, 0.0)
    _zero_acc(zbuf, acc, s)
    plsc.subcore_barrier()
    ebase = wid * ept

    def step(i, _):
        off = ebase + i * EB
        pltpu.sync_copy(row_hbm.at[pl.ds(off, EB)], ridx)
        pltpu.sync_copy(col_hbm.at[pl.ds(off, EB)], cidx)
        pltpu.async_copy(w_hbm.at[ridx], rows_v, sem).wait()
        pltpu.sync_copy(rows_v, acc.at[cidx], add=True)
        return 0

    lax.fori_loop(0, ept // EB, step, 0)
    plsc.subcore_barrier()
    _write_out(acc, out_hbm, c, s)


NBLK = E // NW // EB
CHUNK = 25
NCHUNK = NBLK // CHUNK


NB = 3
LOOK = NB - 1


@functools.cache
def _get_prop2_kernel():
    return functools.partial(
        pl.kernel,
        mesh=_mesh(),
        out_type=jax.ShapeDtypeStruct((NC, N, D), jnp.float32),
        scratch_types=[
            pltpu.VMEM((CHUNK, EB), jnp.int32),
            pltpu.VMEM((CHUNK, EB), jnp.int32),
            pltpu.VMEM((EB, D), jnp.float32),
            pltpu.VMEM((EB, D), jnp.float32),
            pltpu.VMEM((EB, D), jnp.float32),
            pltpu.VMEM_SHARED((N, D), jnp.float32),
            pltpu.SemaphoreType.DMA,
            pltpu.SemaphoreType.DMA,
            pltpu.SemaphoreType.DMA,
            pltpu.SemaphoreType.DMA,
            pltpu.SemaphoreType.DMA,
            pltpu.SemaphoreType.DMA,
        ],
    )(_prop2_body)


def _prop2_body(w_hbm, zeros_hbm, row3_hbm, col3_hbm, out_hbm,
                ridx, cidx, b0, b1, b2, acc, g0, g1, g2, s0, s1, s2):
    c = lax.axis_index("c")
    s = lax.axis_index("s")
    wid = c * NS + s
    _zero_acc_hbm(zeros_hbm, acc, s)
    plsc.subcore_barrier()

    bufs = (b0, b1, b2)
    gsems = (g0, g1, g2)
    ssems = (s0, s1, s2)

    def chunk(ch, _):
        pltpu.sync_copy(row3_hbm.at[wid, ch], ridx)
        pltpu.sync_copy(col3_hbm.at[wid, ch], cidx)
        pend_g = [None] * CHUNK
        pend_s = [None] * NB
        for t in range(CHUNK + LOOK):
            if t < CHUNK:
                b = t % NB
                if pend_s[b] is not None:
                    pend_s[b].wait()
                    pend_s[b] = None
                pend_g[t] = pltpu.async_copy(
                    w_hbm.at[ridx.at[t]], bufs[b], gsems[b])
            if t >= LOOK:
                tt = t - LOOK
                b2_ = tt % NB
                pend_g[tt].wait()
                pend_s[b2_] = pltpu.async_copy(
                    bufs[b2_], acc.at[cidx.at[tt]], ssems[b2_], add=True)
        for b in range(NB):
            if pend_s[b] is not None:
                pend_s[b].wait()
        return 0

    lax.fori_loop(0, NCHUNK, chunk, 0)
    plsc.subcore_barrier()
    _write_out(acc, out_hbm, c, s)


R = 2000
G = N // R
_f32 = jnp.float32


def _dot_t(a, w):
    return lax.dot_general(a, w, (((1,), (1,)), ((), ())),
                           preferred_element_type=_f32,
                           precision=lax.Precision.HIGHEST)


def _prep_body(dp_ref, x_ref, w0_ref, dis_ref, d2_ref):
    deg = dp_ref[0, :, 0:1] + dp_ref[1, :, 0:1]
    dis = jnp.where(deg > 0, lax.rsqrt(jnp.maximum(deg, 1e-12)), 0.0)
    dis_ref[...] = dis
    d2_ref[...] = dis * dis
    w0_ref[...] = x_ref[...] * dis


def _prep(dp, x):
    return pl.pallas_call(
        _prep_body,
        grid=(G,),
        in_specs=[
            pl.BlockSpec((2, R, D), lambda i: (0, i, 0)),
            pl.BlockSpec((R, D), lambda i: (i, 0)),
        ],
        out_specs=[
            pl.BlockSpec((R, D), lambda i: (i, 0)),
            pl.BlockSpec((R, 1), lambda i: (i, 0)),
            pl.BlockSpec((R, 1), lambda i: (i, 0)),
        ],
        out_shape=[
            jax.ShapeDtypeStruct((N, D), _f32),
            jax.ShapeDtypeStruct((N, 1), _f32),
            jax.ShapeDtypeStruct((N, 1), _f32),
        ],
    )(dp, x)


def _step_body(p_ref, dis_ref, d2_ref, w_ref, wn_ref, y_ref):
    b = p_ref[0] + p_ref[1]
    y_ref[...] = _dot_t(b * dis_ref[...], w_ref[...])
    wn_ref[...] = b * d2_ref[...]


def _step(p, dis, d2, w):
    return pl.pallas_call(
        _step_body,
        grid=(G,),
        in_specs=[
            pl.BlockSpec((2, R, D), lambda i: (0, i, 0)),
            pl.BlockSpec((R, 1), lambda i: (i, 0)),
            pl.BlockSpec((R, 1), lambda i: (i, 0)),
            pl.BlockSpec((D, D), lambda i: (0, 0)),
        ],
        out_specs=[
            pl.BlockSpec((R, D), lambda i: (i, 0)),
            pl.BlockSpec((R, D), lambda i: (i, 0)),
        ],
        out_shape=[
            jax.ShapeDtypeStruct((N, D), _f32),
            jax.ShapeDtypeStruct((N, D), _f32),
        ],
    )(p, dis, d2, w)


def _fin1_body(x_ref, w0_ref, y1_ref, y2_ref, y3_ref, b_ref, dis_ref,
               h_ref, wh_ref):
    out = (_dot_t(x_ref[...], w0_ref[...]) + y1_ref[...] + y2_ref[...]
           + y3_ref[...] + b_ref[...])
    h = jnp.maximum(out, 0.0)
    h_ref[...] = h
    wh_ref[...] = h * dis_ref[...]


def _fin1(x, w0, y1, y2, y3, b, dis):
    return pl.pallas_call(
        _fin1_body,
        grid=(G,),
        in_specs=[
            pl.BlockSpec((R, D), lambda i: (i, 0)),
            pl.BlockSpec((D, D), lambda i: (0, 0)),
            pl.BlockSpec((R, D), lambda i: (i, 0)),
            pl.BlockSpec((R, D), lambda i: (i, 0)),
            pl.BlockSpec((R, D), lambda i: (i, 0)),
            pl.BlockSpec((1, D), lambda i: (0, 0)),
            pl.BlockSpec((R, 1), lambda i: (i, 0)),
        ],
        out_specs=[
            pl.BlockSpec((R, D), lambda i: (i, 0)),
            pl.BlockSpec((R, D), lambda i: (i, 0)),
        ],
        out_shape=[
            jax.ShapeDtypeStruct((N, D), _f32),
            jax.ShapeDtypeStruct((N, D), _f32),
        ],
    )(x, w0, y1, y2, y3, b, dis)


def _fin2_body(h_ref, w0_ref, z1_ref, z2_ref, b_ref, ts_ref, tew_ref,
               teb_ref, fwh_ref, fwt_ref, fb_ref, out_ref):
    out2 = jnp.maximum(
        _dot_t(h_ref[...], w0_ref[...]) + z1_ref[...] + z2_ref[...]
        + b_ref[...], 0.0)
    ts_all = ts_ref[...]
    tmin = jnp.min(ts_all)
    tmax = jnp.max(ts_all)
    i = pl.program_id(0)
    tsb = ts_ref[pl.ds(i * R, R), :]
    t = (tsb - tmin) / (tmax - tmin + 1e-8)
    te = t * tew_ref[...] + teb_ref[...]
    res = (lax.dot_general(out2, fwh_ref[...], (((1,), (0,)), ((), ())),
                           preferred_element_type=_f32,
                           precision=lax.Precision.HIGHEST)
           + lax.dot_general(te, fwt_ref[...], (((1,), (0,)), ((), ())),
                             preferred_element_type=_f32,
                             precision=lax.Precision.HIGHEST)
           + fb_ref[...])
    out_ref[...] = res


def _fin2(h, w0, z1, z2, b, ts2, tew, teb, fwh, fwt, fb):
    return pl.pallas_call(
        _fin2_body,
        grid=(G,),
        in_specs=[
            pl.BlockSpec((R, D), lambda i: (i, 0)),
            pl.BlockSpec((D, D), lambda i: (0, 0)),
            pl.BlockSpec((R, D), lambda i: (i, 0)),
            pl.BlockSpec((R, D), lambda i: (i, 0)),
            pl.BlockSpec((1, D), lambda i: (0, 0)),
            pl.BlockSpec((N, 1), lambda i: (0, 0)),
            pl.BlockSpec((1, D), lambda i: (0, 0)),
            pl.BlockSpec((1, D), lambda i: (0, 0)),
            pl.BlockSpec((D, 2), lambda i: (0, 0)),
            pl.BlockSpec((D, 2), lambda i: (0, 0)),
            pl.BlockSpec((1, 2), lambda i: (0, 0)),
        ],
        out_specs=pl.BlockSpec((R, 2), lambda i: (i, 0)),
        out_shape=jax.ShapeDtypeStruct((N, 2), _f32),
    )(h, w0, z1, z2, b, ts2, tew, teb, fwh, fwt, fb)


def kernel(x, edge_index, ts, c1_w0, c1_w1, c1_w2, c1_w3, c1_b,
           c2_w0, c2_w1, c2_w2, c2_b, te_w, te_b, fc_w, fc_b):
    row = edge_index[0]
    col = edge_index[1]

    deg_kernel = _get_deg_kernel()
    prop2 = _get_prop2_kernel()
    row3 = row.reshape(NW, NCHUNK, CHUNK, EB)
    col3 = col.reshape(NW, NCHUNK, CHUNK, EB)
    zeros = jnp.zeros((N, D), jnp.float32)

    def prop_kernel(w, _r, _c):
        return prop2(w, zeros, row3, col3)

    dp = deg_kernel(col3, zeros)
    w0, dis, d2 = _prep(dp, x)

    p1 = prop_kernel(w0, row, col)
    w1, y1 = _step(p1, dis, d2, c1_w1)
    p2 = prop_kernel(w1, row, col)
    w2, y2 = _step(p2, dis, d2, c1_w2)
    p3 = prop_kernel(w2, row, col)
    _, y3 = _step(p3, dis, d2, c1_w3)

    h, wh = _fin1(x, c1_w0, y1, y2, y3, c1_b.reshape(1, D), dis)

    q1 = prop_kernel(wh, row, col)
    v1, z1 = _step(q1, dis, d2, c2_w1)
    q2 = prop_kernel(v1, row, col)
    _, z2 = _step(q2, dis, d2, c2_w2)

    return _fin2(h, c2_w0, z1, z2, c2_b.reshape(1, D), ts.reshape(N, 1),
                 te_w.reshape(1, D), te_b.reshape(1, D),
                 fc_w[:, :D].T, fc_w[:, D:].T, fc_b.reshape(1, 2))
